# baseline (speedup 1.0000x reference)
import functools

import jax
import jax.numpy as jnp
from jax.experimental import pallas as pl

TB = 64
VP = 128


def _selu(x):
    alpha = 1.6732632423543772848170429916717
    scale = 1.0507009873554804934193349852946
    return scale * jnp.where(x > 0, x, alpha * (jnp.exp(jnp.minimum(x, 0.0)) - 1.0))


def _shift(x, k):
    if k == 0:
        return x
    return jnp.concatenate([x[k:], x[:k]], axis=0)


def _branch(ids, e_ref, w0_ref, b0_ref, w1_ref, b1_ref, L, K):
    n = ids.shape[0] * L
    f0 = w0_ref.shape[2]
    f1 = w1_ref.shape[2]
    iota3 = jax.lax.broadcasted_iota(jnp.int32, (ids.shape[0], L, VP), 2)
    oh = (ids[:, :, None] == iota3).astype(jnp.bfloat16).reshape(n, VP)
    acc0 = jnp.zeros((n, f0), jnp.float32)
    for k in range(K):
        tk = jnp.dot(e_ref[...], w0_ref[k],
                     preferred_element_type=jnp.float32).astype(jnp.bfloat16)
        acc0 = acc0 + jnp.dot(_shift(oh, k), tk,
                              preferred_element_type=jnp.float32)
    y0 = _selu(acc0 + b0_ref[...]).astype(jnp.bfloat16)
    acc1 = jnp.zeros((n, f1), jnp.float32)
    for k in range(K):
        acc1 = acc1 + jnp.dot(_shift(y0, k), w1_ref[k],
                              preferred_element_type=jnp.float32)
    y1 = _selu(acc1 + b1_ref[...])
    valid = L - 2 * (K - 1)
    return jnp.max(y1.reshape(ids.shape[0], L, f1)[:, :valid, :], axis=1)


def _body(api_ref, cof_ref, ea_ref, ec_ref,
          wa0_ref, ba0_ref, wa1_ref, ba1_ref,
          wc0_ref, bc0_ref, wc1_ref, bc1_ref,
          wd0_ref, bd0_ref, wd1_ref, bd1_ref, wd2_ref, bd2_ref,
          wh_ref, bh_ref, out_ref, *, L, K):
    a = _branch(api_ref[...], ea_ref, wa0_ref, ba0_ref, wa1_ref, ba1_ref, L, K)
    c = _branch(cof_ref[...], ec_ref, wc0_ref, bc0_ref, wc1_ref, bc1_ref, L, K)
    h = jnp.concatenate([a, c], axis=1).astype(jnp.bfloat16)
    h = jax.nn.relu(jnp.dot(h, wd0_ref[...], preferred_element_type=jnp.float32)
                    + bd0_ref[...]).astype(jnp.bfloat16)
    h = jax.nn.relu(jnp.dot(h, wd1_ref[...], preferred_element_type=jnp.float32)
                    + bd1_ref[...]).astype(jnp.bfloat16)
    h = jax.nn.relu(jnp.dot(h, wd2_ref[...], preferred_element_type=jnp.float32)
                    + bd2_ref[...]).astype(jnp.bfloat16)
    o = jnp.dot(h, wh_ref[...], preferred_element_type=jnp.float32) + bh_ref[...]
    out_ref[...] = jax.nn.sigmoid(o)


def kernel(api, cof, E_api, E_cof, Wa0, ba0, Wa1, ba1, Wc0, bc0, Wc1, bc1,
           Wd0, bd0, Wd1, bd1, Wd2, bd2, Wh, bh):
    B, L = api.shape
    V, D = E_api.shape
    K = Wa0.shape[0]
    H = Wd0.shape[1]

    api32 = api.astype(jnp.int32)
    cof32 = cof.astype(jnp.int32)
    bf = lambda a: a.astype(jnp.bfloat16)
    ea = bf(jnp.zeros((VP, D), jnp.float32).at[:V].set(E_api))
    ec = bf(jnp.zeros((VP, D), jnp.float32).at[:V].set(E_cof))

    full = lambda arr: pl.BlockSpec(arr.shape, lambda i: (0,) * arr.ndim)
    row2 = lambda a: a.reshape(1, -1)

    args = (api32, cof32, ea, ec,
            bf(Wa0), row2(ba0), bf(Wa1), row2(ba1),
            bf(Wc0), row2(bc0), bf(Wc1), row2(bc1),
            bf(Wd0), row2(bd0), bf(Wd1), row2(bd1), bf(Wd2), row2(bd2),
            bf(Wh), row2(bh))
    in_specs = [pl.BlockSpec((TB, L), lambda i: (i, 0)),
                pl.BlockSpec((TB, L), lambda i: (i, 0))]
    in_specs += [full(a) for a in args[2:]]

    return pl.pallas_call(
        functools.partial(_body, L=L, K=K),
        grid=(B // TB,),
        in_specs=in_specs,
        out_specs=pl.BlockSpec((TB, 1), lambda i: (i, 0)),
        out_shape=jax.ShapeDtypeStruct((B, 1), jnp.float32),
    )(*args)


# speedup vs baseline: 1.1725x; 1.1725x over previous
import functools

import jax
import jax.numpy as jnp
from jax.experimental import pallas as pl

TB = 64
VP = 128


def _selu(x):
    alpha = 1.6732632423543772848170429916717
    scale = 1.0507009873554804934193349852946
    return scale * jnp.where(x > 0, x, alpha * (jnp.exp(jnp.minimum(x, 0.0)) - 1.0))


def _shift(x, k):
    if k == 0:
        return x
    return jnp.concatenate([x[k:], x[:k]], axis=0)


def _branch(ids, e_ref, w0_ref, b0_ref, w1_ref, b1_ref, L, K):
    n = ids.shape[0] * L
    f0 = w0_ref.shape[2]
    f1 = w1_ref.shape[2]
    iota3 = jax.lax.broadcasted_iota(jnp.int32, (ids.shape[0], L, VP), 2)
    oh = (ids[:, :, None] == iota3).astype(jnp.bfloat16).reshape(n, VP)
    emb = jnp.dot(oh, e_ref[...],
                  preferred_element_type=jnp.float32).astype(jnp.bfloat16)
    acc0 = jnp.zeros((n, f0), jnp.float32)
    for k in range(K):
        acc0 = acc0 + jnp.dot(_shift(emb, k), w0_ref[k],
                              preferred_element_type=jnp.float32)
    y0 = _selu(acc0 + b0_ref[...]).astype(jnp.bfloat16)
    acc1 = jnp.zeros((n, f1), jnp.float32)
    for k in range(K):
        acc1 = acc1 + jnp.dot(_shift(y0, k), w1_ref[k],
                              preferred_element_type=jnp.float32)
    valid = L - 2 * (K - 1)
    m = jnp.max(acc1.reshape(ids.shape[0], L, f1)[:, :valid, :], axis=1)
    return _selu(m + b1_ref[...])


def _body(api_ref, cof_ref, ea_ref, ec_ref,
          wa0_ref, ba0_ref, wa1_ref, ba1_ref,
          wc0_ref, bc0_ref, wc1_ref, bc1_ref,
          wd0_ref, bd0_ref, wd1_ref, bd1_ref, wd2_ref, bd2_ref,
          wh_ref, bh_ref, out_ref, *, L, K):
    a = _branch(api_ref[...], ea_ref, wa0_ref, ba0_ref, wa1_ref, ba1_ref, L, K)
    c = _branch(cof_ref[...], ec_ref, wc0_ref, bc0_ref, wc1_ref, bc1_ref, L, K)
    h = jnp.concatenate([a, c], axis=1).astype(jnp.bfloat16)
    h = jax.nn.relu(jnp.dot(h, wd0_ref[...], preferred_element_type=jnp.float32)
                    + bd0_ref[...]).astype(jnp.bfloat16)
    h = jax.nn.relu(jnp.dot(h, wd1_ref[...], preferred_element_type=jnp.float32)
                    + bd1_ref[...]).astype(jnp.bfloat16)
    h = jax.nn.relu(jnp.dot(h, wd2_ref[...], preferred_element_type=jnp.float32)
                    + bd2_ref[...]).astype(jnp.bfloat16)
    o = jnp.dot(h, wh_ref[...], preferred_element_type=jnp.float32) + bh_ref[...]
    out_ref[...] = jax.nn.sigmoid(o)


def kernel(api, cof, E_api, E_cof, Wa0, ba0, Wa1, ba1, Wc0, bc0, Wc1, bc1,
           Wd0, bd0, Wd1, bd1, Wd2, bd2, Wh, bh):
    B, L = api.shape
    V, D = E_api.shape
    K = Wa0.shape[0]
    H = Wd0.shape[1]

    api32 = api.astype(jnp.int32)
    cof32 = cof.astype(jnp.int32)
    bf = lambda a: a.astype(jnp.bfloat16)
    ea = bf(jnp.zeros((VP, D), jnp.float32).at[:V].set(E_api))
    ec = bf(jnp.zeros((VP, D), jnp.float32).at[:V].set(E_cof))

    full = lambda arr: pl.BlockSpec(arr.shape, lambda i: (0,) * arr.ndim)
    row2 = lambda a: a.reshape(1, -1)

    args = (api32, cof32, ea, ec,
            bf(Wa0), row2(ba0), bf(Wa1), row2(ba1),
            bf(Wc0), row2(bc0), bf(Wc1), row2(bc1),
            bf(Wd0), row2(bd0), bf(Wd1), row2(bd1), bf(Wd2), row2(bd2),
            bf(Wh), row2(bh))
    in_specs = [pl.BlockSpec((TB, L), lambda i: (i, 0)),
                pl.BlockSpec((TB, L), lambda i: (i, 0))]
    in_specs += [full(a) for a in args[2:]]

    return pl.pallas_call(
        functools.partial(_body, L=L, K=K),
        grid=(B // TB,),
        in_specs=in_specs,
        out_specs=pl.BlockSpec((TB, 1), lambda i: (i, 0)),
        out_shape=jax.ShapeDtypeStruct((B, 1), jnp.float32),
    )(*args)
